# Initial kernel scaffold; baseline (speedup 1.0000x reference)
import jax
import jax.numpy as jnp
from jax.experimental import pallas as pl


def kernel(x, edge_index, edge_attr, batch, edge_emb, agg_weights, mlp1, pre_nn, post_nn, bn_gamma, bn_beta, mlp2, mlp3):
    raise NotImplementedError("write your pallas kernel here")


# speedup vs baseline: 1.2312x; 1.2312x over previous
import functools
import jax
import jax.numpy as jnp
from jax.experimental import pallas as pl
from jax.experimental.pallas import tpu as pltpu

F = 1262
FP = 1280
N = 10000
NP = 10240
E = 40000
EP = 40960
NG = 16
RB = 256


def _pad2(a, r, c):
    return jnp.pad(a, ((0, r - a.shape[0]), (0, c - a.shape[1])))


def _pad1(a, n):
    return jnp.pad(a, ((0, n - a.shape[0]),))


def _dot(a, b):
    return jnp.dot(a, b, preferred_element_type=jnp.float32)


def _node_body(x_ref, w1, b1, wd, b0, ws, x1_out, p_out, q_out):
    x1 = jnp.maximum(_dot(x_ref[...], w1[...]) + b1[...], 0.0)
    x1_out[...] = x1
    p_out[...] = _dot(x1, wd[...]) + b0[...]
    q_out[...] = _dot(x1, ws[...])


def _node_stage(xp, w1, b1, wd, b0, ws):
    nblk = NP // RB
    full = pl.BlockSpec((FP, FP), lambda i: (0, 0))
    brow = pl.BlockSpec((1, FP), lambda i: (0, 0))
    blk = pl.BlockSpec((RB, FP), lambda i: (i, 0))
    return pl.pallas_call(
        _node_body,
        grid=(nblk,),
        in_specs=[blk, full, brow, full, brow, full],
        out_specs=[blk, blk, blk],
        out_shape=[jax.ShapeDtypeStruct((NP, FP), jnp.float32)] * 3,
    )(xp, w1, b1, wd, b0, ws)


def _edge_mlp_body(g_ref, w1, b1, w2, b2, w3, b3, w4, b4, h_out):
    h = g_ref[...]
    h = jnp.maximum(_dot(h, w1[...]) + b1[...], 0.0)
    h = jnp.maximum(_dot(h, w2[...]) + b2[...], 0.0)
    h = jnp.maximum(_dot(h, w3[...]) + b3[...], 0.0)
    h_out[...] = _dot(h, w4[...]) + b4[...]


def _edge_mlp(g, ws):
    nblk = EP // RB
    full = pl.BlockSpec((FP, FP), lambda i: (0, 0))
    brow = pl.BlockSpec((1, FP), lambda i: (0, 0))
    blk = pl.BlockSpec((RB, FP), lambda i: (i, 0))
    args = []
    for (w, b) in ws:
        args += [w, b]
    return pl.pallas_call(
        _edge_mlp_body,
        grid=(nblk,),
        in_specs=[blk] + [full, brow] * 4,
        out_specs=blk,
        out_shape=jax.ShapeDtypeStruct((EP, FP), jnp.float32),
    )(g, *args)


def _post_body(x1_ref, agg_ref, wx, wa, b0, w1, b1, w2, b2, w3, b3, w4, b4,
               out_ref, ps_ref, pq_ref):
    i = pl.program_id(0)
    h = _dot(x1_ref[...], wx[...]) + _dot(agg_ref[...], wa[...]) + b0[...]
    h = jnp.maximum(h, 0.0)
    h = jnp.maximum(_dot(h, w1[...]) + b1[...], 0.0)
    h = jnp.maximum(_dot(h, w2[...]) + b2[...], 0.0)
    h = jnp.maximum(_dot(h, w3[...]) + b3[...], 0.0)
    h = _dot(h, w4[...]) + b4[...]
    out_ref[...] = h
    rows = jax.lax.broadcasted_iota(jnp.int32, (RB, 1), 0) + i * RB
    m = (rows < N).astype(jnp.float32)
    hm = h * m
    ps = jnp.sum(hm.reshape(RB // 8, 8, FP), axis=0)
    pq = jnp.sum((hm * hm).reshape(RB // 8, 8, FP), axis=0)

    @pl.when(i == 0)
    def _():
        ps_ref[...] = jnp.zeros_like(ps_ref)
        pq_ref[...] = jnp.zeros_like(pq_ref)

    ps_ref[...] += ps
    pq_ref[...] += pq


def _post_stage(x1, agg, ws):
    nblk = NP // RB
    full = pl.BlockSpec((FP, FP), lambda i: (0, 0))
    brow = pl.BlockSpec((1, FP), lambda i: (0, 0))
    blk = pl.BlockSpec((RB, FP), lambda i: (i, 0))
    acc = pl.BlockSpec((8, FP), lambda i: (0, 0))
    args = []
    for (w, b) in ws[1:]:
        args += [w, b]
    return pl.pallas_call(
        _post_body,
        grid=(nblk,),
        in_specs=[blk, blk, full, full, brow] + [full, brow] * 4,
        out_specs=[blk, acc, acc],
        out_shape=[jax.ShapeDtypeStruct((NP, FP), jnp.float32),
                   jax.ShapeDtypeStruct((8, FP), jnp.float32),
                   jax.ShapeDtypeStruct((8, FP), jnp.float32)],
    )(x1, agg, ws[0][0], ws[0][1], ws[0][2], *args)


def _bn_force_body(out_ref, ps_ref, pq_ref, gam, bet, oh_ref,
                   w1, b1, w2, b2, w3, b3, force_ref, pool_ref):
    i = pl.program_id(0)
    mu = jnp.sum(ps_ref[...], axis=0, keepdims=True) / N
    var = jnp.sum(pq_ref[...], axis=0, keepdims=True) / N - mu * mu
    h = (out_ref[...] - mu) * jax.lax.rsqrt(var + 1e-5) * gam[...] + bet[...]
    h = jnp.maximum(h, 0.0)
    part = jax.lax.dot_general(oh_ref[...], h, (((0,), (0,)), ((), ())),
                               preferred_element_type=jnp.float32)

    @pl.when(i == 0)
    def _():
        pool_ref[...] = jnp.zeros_like(pool_ref)

    pool_ref[...] += part
    f = jnp.maximum(_dot(h, w1[...]) + b1[...], 0.0)
    f = jnp.maximum(_dot(f, w2[...]) + b2[...], 0.0)
    force_ref[...] = _dot(f, w3[...]) + b3[...]


def _bn_force_stage(out, ps, pq, gam, bet, ohp, m3):
    nblk = NP // RB
    blk = pl.BlockSpec((RB, FP), lambda i: (i, 0))
    acc8 = pl.BlockSpec((8, FP), lambda i: (0, 0))
    brow = pl.BlockSpec((1, FP), lambda i: (0, 0))
    bblk = pl.BlockSpec((RB, 128), lambda i: (i, 0))
    poolspec = pl.BlockSpec((128, FP), lambda i: (0, 0))
    (w1, b1), (w2, b2), (w3, b3) = m3
    h1, h2, h3 = w1.shape[1], w2.shape[1], w3.shape[1]
    specs = [blk, acc8, acc8, brow, brow, bblk,
             pl.BlockSpec((FP, h1), lambda i: (0, 0)),
             pl.BlockSpec((1, h1), lambda i: (0, 0)),
             pl.BlockSpec((h1, h2), lambda i: (0, 0)),
             pl.BlockSpec((1, h2), lambda i: (0, 0)),
             pl.BlockSpec((h2, h3), lambda i: (0, 0)),
             pl.BlockSpec((1, h3), lambda i: (0, 0))]
    return pl.pallas_call(
        _bn_force_body,
        grid=(nblk,),
        in_specs=specs,
        out_specs=[pl.BlockSpec((RB, h3), lambda i: (i, 0)), poolspec],
        out_shape=[jax.ShapeDtypeStruct((NP, h3), jnp.float32),
                   jax.ShapeDtypeStruct((128, FP), jnp.float32)],
    )(out, ps, pq, gam, bet, ohp, w1, b1, w2, b2, w3, b3)


def _energy_body(pool_ref, w1, b1, w2, b2, w3, b3, e_ref):
    f = jnp.maximum(_dot(pool_ref[...], w1[...]) + b1[...], 0.0)
    f = jnp.maximum(_dot(f, w2[...]) + b2[...], 0.0)
    e_ref[...] = _dot(f, w3[...]) + b3[...]


def _energy_stage(pool, m2):
    (w1, b1), (w2, b2), (w3, b3) = m2
    h1, h2, h3 = w1.shape[1], w2.shape[1], w3.shape[1]
    full = lambda a: pl.BlockSpec(a.shape, lambda: tuple(0 for _ in a.shape))
    return pl.pallas_call(
        _energy_body,
        in_specs=[full(pool), full(w1), full(b1), full(w2), full(b2),
                  full(w3), full(b3)],
        out_specs=pl.BlockSpec((128, h3), lambda: (0, 0)),
        out_shape=jax.ShapeDtypeStruct((128, h3), jnp.float32),
    )(pool, w1, b1, w2, b2, w3, b3)


def _eemb_body(emb_ref, we_ref, out_ref):
    out_ref[...] = _dot(emb_ref[...], we_ref[...])


def _eemb_stage(embp, wep):
    return pl.pallas_call(
        _eemb_body,
        in_specs=[pl.BlockSpec(embp.shape, lambda: (0, 0)),
                  pl.BlockSpec(wep.shape, lambda: (0, 0))],
        out_specs=pl.BlockSpec((embp.shape[0], FP), lambda: (0, 0)),
        out_shape=jax.ShapeDtypeStruct((embp.shape[0], FP), jnp.float32),
    )(embp, wep)


def kernel(x, edge_index, edge_attr, batch, edge_emb, agg_weights,
           mlp1, pre_nn, post_nn, bn_gamma, bn_beta, mlp2, mlp3):
    xp = _pad2(x, NP, FP)
    w1p = _pad2(mlp1[0][0], FP, FP)
    b1p = _pad1(mlp1[0][1], FP)[None, :]

    w0 = pre_nn[0][0]
    wd = _pad2(w0[:F], FP, FP)
    ws = _pad2(w0[F:2 * F], FP, FP)
    we = w0[2 * F:]
    b0 = _pad1(pre_nn[0][1], FP)[None, :]

    x1, P, Q = _node_stage(xp, w1p, b1p, wd, b0, ws)

    ed = edge_emb.shape[1]
    embp = _pad2(edge_emb, 32, 16)
    wep = _pad2(we, 16, FP)
    Eemb = _eemb_stage(embp, wep)

    src = edge_index[0]
    dst = edge_index[1]
    dstp = jnp.concatenate([dst, jnp.full((EP - E,), NP - 1, jnp.int32)])
    srcp = jnp.concatenate([src, jnp.zeros((EP - E,), jnp.int32)])
    attrp = jnp.concatenate([edge_attr, jnp.zeros((EP - E,), jnp.int32)])

    g = jnp.maximum(P[dstp] + Q[srcp] + Eemb[attrp], 0.0)

    pre_ws = [(_pad2(w, FP, FP), _pad1(b, FP)[None, :]) for (w, b) in pre_nn[1:]]
    h = _edge_mlp(g, pre_ws)

    w = jax.nn.softmax(agg_weights)
    cnt = jax.ops.segment_sum(jnp.ones((EP,), jnp.float32), dstp, NP)
    cntc = jnp.maximum(cnt, 1.0)[:, None]
    s = jax.ops.segment_sum(h, dstp, NP)
    mean = s / cntc
    mn = jax.ops.segment_min(h, dstp, NP)
    mn = jnp.where(cnt[:, None] > 0, mn, 0.0)
    mx = jax.ops.segment_max(h, dstp, NP)
    mx = jnp.where(cnt[:, None] > 0, mx, 0.0)
    sq = jax.ops.segment_sum(h * h, dstp, NP) / cntc
    std = jnp.sqrt(jax.nn.relu(sq - mean ** 2) + 1e-5)
    agg = w[0] * s + w[1] * mean + w[2] * mn + w[3] * mx + w[4] * std

    pw0 = post_nn[0][0]
    wx = _pad2(pw0[:F], FP, FP)
    wa = _pad2(pw0[F:], FP, FP)
    pb0 = _pad1(post_nn[0][1], FP)[None, :]
    post_ws = [(wx, wa, pb0)] + [(_pad2(w_, FP, FP), _pad1(b_, FP)[None, :])
                                 for (w_, b_) in post_nn[1:]]
    out, ps, pq = _post_stage(x1, agg, post_ws)

    gam = _pad1(bn_gamma, FP)[None, :]
    bet = _pad1(bn_beta, FP)[None, :]
    batchp = jnp.concatenate([batch, jnp.full((NP - N,), NG, jnp.int32)])
    ohp = (batchp[:, None] == jnp.arange(128)[None, :]).astype(jnp.float32)

    def padmlp(m):
        dims = [FP] + [((w_.shape[1] + 127) // 128) * 128 for (w_, _) in m]
        return [(_pad2(w_, dims[i], dims[i + 1]),
                 _pad1(b_, dims[i + 1])[None, :]) for i, (w_, b_) in enumerate(m)]

    m3 = padmlp(mlp3)
    force_p, pool = _bn_force_stage(out, ps, pq, gam, bet, ohp, m3)
    m2 = padmlp(mlp2)
    energy_p = _energy_stage(pool, m2)

    force = force_p[:N, :3]
    energy = energy_p[:NG, :1]
    return force, energy, jnp.float32(1.0)
